# baseline (speedup 1.0000x reference)
import functools

import jax
import jax.numpy as jnp
from jax import lax
from jax.experimental import pallas as pl
from jax.experimental.pallas import tpu as pltpu
from jax.experimental.pallas import tpu_sc as plsc

NUM_NEWS = 1000000
NUM_CATEGORIES = 1000
NEWS_DIM = 64
CATEGORY_DIM = 16
BATCH = 4096
SEQ_LEN = 50
TOTAL = BATCH * SEQ_LEN
JOINT_DIM = NEWS_DIM + CATEGORY_DIM
ROW_PAD = 128

NUM_CORES = 2
NUM_SUBCORES = 16
NW = NUM_CORES * NUM_SUBCORES
PER_W = TOTAL // NW
CHUNK = 128
N_CHUNK = PER_W // CHUNK
LANES = 16


def _sc_body(nidx2_hbm, nidx_hbm, cidx_hbm, news_hbm, cat_hbm, out_hbm,
             nidx2_v, nidx_v, cidx_v, cat_v, pair_v, stage_v, sem):
    cid = lax.axis_index("c")
    sid = lax.axis_index("s")
    wid = sid * NUM_CORES + cid
    base_row = wid * PER_W
    pltpu.sync_copy(nidx2_hbm.at[pl.ds(base_row, PER_W)], nidx2_v)
    pltpu.sync_copy(nidx_hbm.at[pl.ds(base_row, PER_W)], nidx_v)
    pltpu.sync_copy(cidx_hbm.at[pl.ds(base_row, PER_W)], cidx_v)
    pltpu.sync_copy(cat_hbm, cat_v)

    def chunk_body(c, carry):
        row0 = base_row + c * CHUNK
        idx_n = nidx2_v.at[pl.ds(c * CHUNK, CHUNK)]
        pltpu.async_copy(news_hbm.at[idx_n], pair_v, sem)
        pltpu.make_async_copy(news_hbm.at[idx_n], pair_v, sem).wait()

        def group_body(g, carry2):
            ids = nidx_v[pl.ds(c * CHUNK + g * LANES, LANES)]
            cids = cidx_v[pl.ds(c * CHUNK + g * LANES, LANES)]
            rows = g * LANES + lax.iota(jnp.int32, LANES)
            half = (ids & 1) * NEWS_DIM
            dst0 = rows * JOINT_DIM
            for col in range(NEWS_DIM):
                vals = plsc.load_gather(pair_v, [rows, half + col])
                plsc.store_scatter(stage_v, [dst0 + col], vals)
            caddr = cids * CATEGORY_DIM
            for col in range(CATEGORY_DIM):
                vals = plsc.load_gather(cat_v, [caddr + col])
                plsc.store_scatter(stage_v, [dst0 + (NEWS_DIM + col)], vals)
            return carry2

        lax.fori_loop(0, CHUNK // LANES, group_body, 0)
        pltpu.sync_copy(stage_v, out_hbm.at[pl.ds(row0 * JOINT_DIM,
                                                  CHUNK * JOINT_DIM)])
        return carry

    lax.fori_loop(0, N_CHUNK, chunk_body, 0)


@jax.jit
def _joint_embed(news_idx2, news_idx, cat_idx, news128, cat_flat):
    mesh = plsc.VectorSubcoreMesh(core_axis_name="c", subcore_axis_name="s")
    f = functools.partial(
        pl.kernel,
        mesh=mesh,
        out_type=jax.ShapeDtypeStruct((TOTAL * JOINT_DIM,), jnp.float32),
        scratch_types=[
            pltpu.VMEM((PER_W,), jnp.int32),
            pltpu.VMEM((PER_W,), jnp.int32),
            pltpu.VMEM((PER_W,), jnp.int32),
            pltpu.VMEM((NUM_CATEGORIES * CATEGORY_DIM,), jnp.float32),
            pltpu.VMEM((CHUNK, ROW_PAD), jnp.float32),
            pltpu.VMEM((CHUNK * JOINT_DIM,), jnp.float32),
            pltpu.SemaphoreType.DMA,
        ],
        compiler_params=pltpu.CompilerParams(needs_layout_passes=False),
    )(_sc_body)
    return f(news_idx2, news_idx, cat_idx, news128, cat_flat)


def kernel(news_ids, category_ids, news_table, category_table):
    news_idx = news_ids.reshape(TOTAL)
    news_idx2 = news_idx >> 1
    cat_idx = category_ids.reshape(TOTAL)
    news128 = news_table.reshape(NUM_NEWS // 2, ROW_PAD)
    cat_flat = category_table.reshape(NUM_CATEGORIES * CATEGORY_DIM)
    out = _joint_embed(news_idx2, news_idx, cat_idx, news128, cat_flat)
    return out.reshape(BATCH, SEQ_LEN, JOINT_DIM)


# speedup vs baseline: 1.0758x; 1.0758x over previous
import functools

import jax
import jax.numpy as jnp
from jax import lax
from jax.experimental import pallas as pl
from jax.experimental.pallas import tpu as pltpu
from jax.experimental.pallas import tpu_sc as plsc

NUM_NEWS = 1000000
NUM_CATEGORIES = 1000
NEWS_DIM = 64
CATEGORY_DIM = 16
BATCH = 4096
SEQ_LEN = 50
TOTAL = BATCH * SEQ_LEN
JOINT_DIM = NEWS_DIM + CATEGORY_DIM
ROW_PAD = 128

NUM_CORES = 2
NUM_SUBCORES = 16
NW = NUM_CORES * NUM_SUBCORES
PER_W = TOTAL // NW
CHUNK = 128
N_CHUNK = PER_W // CHUNK
LANES = 16
NBUF = 2


def _sc_body(nidx2_hbm, nidx_hbm, cidx_hbm, news_hbm, cat_hbm, out_hbm,
             nidx2_v, nidx_v, cidx_v, cat_v, pair0_v, pair1_v,
             stage0_v, stage1_v, gsem0, gsem1, wsem0, wsem1):
    cid = lax.axis_index("c")
    sid = lax.axis_index("s")
    wid = sid * NUM_CORES + cid
    base_row = wid * PER_W
    pltpu.sync_copy(nidx2_hbm.at[pl.ds(base_row, PER_W)], nidx2_v)
    pltpu.sync_copy(nidx_hbm.at[pl.ds(base_row, PER_W)], nidx_v)
    pltpu.sync_copy(cidx_hbm.at[pl.ds(base_row, PER_W)], cidx_v)
    pltpu.sync_copy(cat_hbm, cat_v)

    pairs = (pair0_v, pair1_v)
    stages = (stage0_v, stage1_v)
    gsems = (gsem0, gsem1)
    wsems = (wsem0, wsem1)

    def gather_copy(c, b):
        idx_n = nidx2_v.at[pl.ds(c * CHUNK, CHUNK)]
        return pltpu.make_async_copy(news_hbm.at[idx_n], pairs[b], gsems[b])

    def write_copy(c, b):
        off = (base_row + c * CHUNK) * JOINT_DIM
        return pltpu.make_async_copy(stages[b],
                                     out_hbm.at[pl.ds(off, CHUNK * JOINT_DIM)],
                                     wsems[b])

    def merge(c, b):
        pv = pairs[b]
        sv = stages[b]
        iota = lax.iota(jnp.int32, LANES)

        def group_body(g, carry):
            base = c * CHUNK + g * LANES
            ids = nidx_v[pl.ds(base, LANES)]
            cids = cidx_v[pl.ds(base, LANES)]
            rows = g * LANES + iota
            src = (ids & 1) * NEWS_DIM
            dst = rows * JOINT_DIM
            for col in range(NEWS_DIM):
                vals = plsc.load_gather(pv, [rows, src])
                plsc.store_scatter(sv, [dst], vals)
                src = src + 1
                dst = dst + 1
            caddr = cids * CATEGORY_DIM
            for col in range(CATEGORY_DIM):
                vals = plsc.load_gather(cat_v, [caddr])
                plsc.store_scatter(sv, [dst], vals)
                caddr = caddr + 1
                dst = dst + 1
            return carry

        lax.fori_loop(0, CHUNK // LANES, group_body, 0)

    gather_copy(0, 0).start()

    def pair_body(g, carry):
        for b in range(NBUF):
            c = g * NBUF + b
            nc = c + 1
            @pl.when(nc < N_CHUNK)
            def _():
                gather_copy(nc, (b + 1) % NBUF).start()
            gather_copy(c, b).wait()
            @pl.when(c >= NBUF)
            def _():
                write_copy(c - NBUF, b).wait()
            merge(c, b)
            write_copy(c, b).start()
        return carry

    lax.fori_loop(0, N_CHUNK // NBUF, pair_body, 0)
    write_copy(N_CHUNK - 2, 0).wait()
    write_copy(N_CHUNK - 1, 1).wait()


@jax.jit
def _joint_embed(news_idx2, news_idx, cat_idx, news128, cat_flat):
    mesh = plsc.VectorSubcoreMesh(core_axis_name="c", subcore_axis_name="s")
    f = functools.partial(
        pl.kernel,
        mesh=mesh,
        out_type=jax.ShapeDtypeStruct((TOTAL * JOINT_DIM,), jnp.float32),
        scratch_types=[
            pltpu.VMEM((PER_W,), jnp.int32),
            pltpu.VMEM((PER_W,), jnp.int32),
            pltpu.VMEM((PER_W,), jnp.int32),
            pltpu.VMEM((NUM_CATEGORIES * CATEGORY_DIM,), jnp.float32),
            pltpu.VMEM((CHUNK, ROW_PAD), jnp.float32),
            pltpu.VMEM((CHUNK, ROW_PAD), jnp.float32),
            pltpu.VMEM((CHUNK * JOINT_DIM,), jnp.float32),
            pltpu.VMEM((CHUNK * JOINT_DIM,), jnp.float32),
            pltpu.SemaphoreType.DMA,
            pltpu.SemaphoreType.DMA,
            pltpu.SemaphoreType.DMA,
            pltpu.SemaphoreType.DMA,
        ],
        compiler_params=pltpu.CompilerParams(needs_layout_passes=False),
    )(_sc_body)
    return f(news_idx2, news_idx, cat_idx, news128, cat_flat)


def kernel(news_ids, category_ids, news_table, category_table):
    news_idx = news_ids.reshape(TOTAL)
    news_idx2 = news_idx >> 1
    cat_idx = category_ids.reshape(TOTAL)
    news128 = lax.bitcast_convert_type(
        lax.bitcast_convert_type(news_table, jnp.uint32).reshape(
            NUM_NEWS // 2, ROW_PAD),
        jnp.float32)
    cat_flat = category_table.reshape(NUM_CATEGORIES * CATEGORY_DIM)
    out = _joint_embed(news_idx2, news_idx, cat_idx, news128, cat_flat)
    return out.reshape(BATCH, SEQ_LEN, JOINT_DIM)


# speedup vs baseline: 1.3138x; 1.2212x over previous
import functools

import jax
import jax.numpy as jnp
from jax import lax
from jax.experimental import pallas as pl
from jax.experimental.pallas import tpu as pltpu
from jax.experimental.pallas import tpu_sc as plsc

NUM_NEWS = 1000000
NUM_CATEGORIES = 1000
NEWS_DIM = 64
CATEGORY_DIM = 16
BATCH = 4096
SEQ_LEN = 50
TOTAL = BATCH * SEQ_LEN
JOINT_DIM = NEWS_DIM + CATEGORY_DIM
ROW_PAD = 128

NUM_CORES = 2
NUM_SUBCORES = 16
NW = NUM_CORES * NUM_SUBCORES
PER_W = TOTAL // NW
CHUNK = 128
N_CHUNK = PER_W // CHUNK
LANES = 16
NBUF = 2


def _sc_body(nidx2_hbm, nidx_hbm, cidx_hbm, news_hbm, cat_hbm, out_hbm,
             nidx2_v, nidx_v, cidx_v, cat_v, pair0_v, pair1_v,
             stage0_v, stage1_v, gsem0, gsem1, wsem0, wsem1):
    cid = lax.axis_index("c")
    sid = lax.axis_index("s")
    wid = sid * NUM_CORES + cid
    base_row = wid * PER_W
    pltpu.sync_copy(nidx2_hbm.at[pl.ds(base_row, PER_W)], nidx2_v)
    pltpu.sync_copy(nidx_hbm.at[pl.ds(base_row, PER_W)], nidx_v)
    pltpu.sync_copy(cidx_hbm.at[pl.ds(base_row, PER_W)], cidx_v)
    pltpu.sync_copy(cat_hbm, cat_v)

    pairs = (pair0_v, pair1_v)
    stages = (stage0_v, stage1_v)
    gsems = (gsem0, gsem1)
    wsems = (wsem0, wsem1)

    def gather_copy(c, b):
        idx_n = nidx2_v.at[pl.ds(c * CHUNK, CHUNK)]
        return pltpu.make_async_copy(news_hbm.at[idx_n], pairs[b], gsems[b])

    def write_copy(c, b):
        off = (base_row + c * CHUNK) * JOINT_DIM
        return pltpu.make_async_copy(stages[b],
                                     out_hbm.at[pl.ds(off, CHUNK * JOINT_DIM)],
                                     wsems[b])

    def merge(c, b):
        pv = pairs[b]
        sv = stages[b]
        iota = lax.iota(jnp.int32, LANES)

        def row_body(r, carry):
            rsplat = jnp.full((LANES,), c * CHUNK + r, jnp.int32)
            idv = plsc.load_gather(nidx_v, [rsplat])
            odd = (idv & 1) != 0
            cidv = plsc.load_gather(cidx_v, [rsplat])
            dbase = r * JOINT_DIM
            for k in range(NEWS_DIM // LANES):
                lo = pv[r, pl.ds(k * LANES, LANES)]
                hi = pv[r, pl.ds(NEWS_DIM + k * LANES, LANES)]
                sv[pl.ds(dbase + k * LANES, LANES)] = jnp.where(odd, hi, lo)
            cvals = plsc.load_gather(cat_v, [cidv * CATEGORY_DIM + iota])
            sv[pl.ds(dbase + NEWS_DIM, LANES)] = cvals
            return carry

        lax.fori_loop(0, CHUNK, row_body, 0)

    gather_copy(0, 0).start()

    def pair_body(g, carry):
        for b in range(NBUF):
            c = g * NBUF + b
            nc = c + 1
            @pl.when(nc < N_CHUNK)
            def _():
                gather_copy(nc, (b + 1) % NBUF).start()
            gather_copy(c, b).wait()
            @pl.when(c >= NBUF)
            def _():
                write_copy(c - NBUF, b).wait()
            merge(c, b)
            write_copy(c, b).start()
        return carry

    lax.fori_loop(0, N_CHUNK // NBUF, pair_body, 0)
    write_copy(N_CHUNK - 2, 0).wait()
    write_copy(N_CHUNK - 1, 1).wait()


@jax.jit
def _joint_embed(news_idx2, news_idx, cat_idx, news128, cat_flat):
    mesh = plsc.VectorSubcoreMesh(core_axis_name="c", subcore_axis_name="s")
    f = functools.partial(
        pl.kernel,
        mesh=mesh,
        out_type=jax.ShapeDtypeStruct((TOTAL * JOINT_DIM,), jnp.float32),
        scratch_types=[
            pltpu.VMEM((PER_W,), jnp.int32),
            pltpu.VMEM((PER_W,), jnp.int32),
            pltpu.VMEM((PER_W,), jnp.int32),
            pltpu.VMEM((NUM_CATEGORIES * CATEGORY_DIM,), jnp.float32),
            pltpu.VMEM((CHUNK, ROW_PAD), jnp.float32),
            pltpu.VMEM((CHUNK, ROW_PAD), jnp.float32),
            pltpu.VMEM((CHUNK * JOINT_DIM,), jnp.float32),
            pltpu.VMEM((CHUNK * JOINT_DIM,), jnp.float32),
            pltpu.SemaphoreType.DMA,
            pltpu.SemaphoreType.DMA,
            pltpu.SemaphoreType.DMA,
            pltpu.SemaphoreType.DMA,
        ],
        compiler_params=pltpu.CompilerParams(needs_layout_passes=False),
    )(_sc_body)
    return f(news_idx2, news_idx, cat_idx, news128, cat_flat)


def kernel(news_ids, category_ids, news_table, category_table):
    news_idx = news_ids.reshape(TOTAL)
    news_idx2 = news_idx >> 1
    cat_idx = category_ids.reshape(TOTAL)
    news128 = lax.bitcast_convert_type(
        lax.bitcast_convert_type(news_table, jnp.uint32).reshape(
            NUM_NEWS // 2, ROW_PAD),
        jnp.float32)
    cat_flat = category_table.reshape(NUM_CATEGORIES * CATEGORY_DIM)
    out = _joint_embed(news_idx2, news_idx, cat_idx, news128, cat_flat)
    return out.reshape(BATCH, SEQ_LEN, JOINT_DIM)
